# Initial kernel scaffold; baseline (speedup 1.0000x reference)
import jax
import jax.numpy as jnp
from jax.experimental import pallas as pl


def kernel(inputs, train_mask, tr_mask, tcl_mask, radii_map, sin_map, cos_map, kernel_mask, border_mask):
    raise NotImplementedError("write your pallas kernel here")


# speedup vs baseline: 17.2636x; 17.2636x over previous
import functools

import jax
import jax.numpy as jnp
from jax import lax
from jax.experimental import pallas as pl
from jax.experimental.pallas import tpu as pltpu
from jax.experimental.pallas import tpu_sc as plsc

B, H, W = 4, 512, 512
N = B * H * W
BH = 64
NH = H // BH
ACC_L = 128

NBINS = 1024
CEMAX = 24.0
SCALE = NBINS / CEMAX
NW = 32
LANES = 16
CHUNK = N // NW


def _tc_body(inp_ref, train_ref, tr_ref, tcl_ref, radii_ref, sin_ref,
             cos_ref, km_ref, bm_ref, acc_ref, ce_ref):
    h = pl.program_id(1)

    x0 = inp_ref[0, 0]
    x1 = inp_ref[0, 1]
    x2 = inp_ref[0, 2]
    x3 = inp_ref[0, 3]
    x4 = inp_ref[0, 4]
    x5 = inp_ref[0, 5]
    x6 = inp_ref[0, 6]
    x7 = inp_ref[0, 7]
    train = train_ref[0]
    tr = tr_ref[0]
    tcl = tcl_ref[0]

    pos = (tr * train) > 0
    posf = pos.astype(jnp.float32)
    neg = ((1 - tr) * train) > 0
    negf = neg.astype(jnp.float32)

    def ce(a, b, t):
        mx = jnp.maximum(a, b)
        lse = mx + jnp.log(jnp.exp(a - mx) + jnp.exp(b - mx))
        return lse - jnp.where(t > 0, b, a)

    ce_tr = ce(x0, x1, tr)
    ce_tcl = ce(x2, x3, tcl)

    n_pos = jnp.sum(posf)
    loss_pos = jnp.sum(ce_tr * posf)
    n_neg = jnp.sum(negf)
    sum_tcl = jnp.sum(ce_tcl * posf)

    tclposf = ((tcl * train) > 0).astype(jnp.float32)
    n_tclpos = jnp.sum(tclposf)

    inv = lax.rsqrt(x4 * x4 + x5 * x5 + 1e-8)
    sp = x4 * inv
    cp = x5 * inv

    def sl1(p, t):
        d = jnp.abs(p - t)
        return jnp.where(d < (1.0 / 9.0), 4.5 * d * d, d - (0.5 / 9.0))

    l1r = jnp.sum(sl1(x6, radii_ref[0]) * tclposf)
    l1s = jnp.sum(sl1(sp, sin_ref[0]) * tclposf)
    l1c = jnp.sum(sl1(cp, cos_ref[0]) * tclposf)

    sig = 1.0 / (1.0 + jnp.exp(-x7))
    mb = ((train * bm_ref[0]) > 0).astype(jnp.float32)
    tgt = (km_ref[0] > 0).astype(jnp.float32)
    da = jnp.sum(sig * tgt * mb)
    db = jnp.sum(sig * sig * mb)
    dc = jnp.sum(tgt * mb)

    ce_ref[0] = jnp.where(neg, ce_tr, -1.0)

    ii = lax.broadcasted_iota(jnp.int32, (1, 1, ACC_L), 2)
    vals = (n_pos, loss_pos, n_neg, sum_tcl, n_tclpos, l1r, l1s, l1c,
            da, db, dc)
    vec = jnp.zeros((1, 1, ACC_L), jnp.float32)
    for q, v in enumerate(vals):
        vec = vec + jnp.where(ii == q, v, 0.0)

    @pl.when(h == 0)
    def _():
        acc_ref[...] = vec

    @pl.when(h != 0)
    def _():
        acc_ref[...] = acc_ref[...] + vec


def _tc_call(inputs, train_mask, tr_mask, tcl_mask, radii_map, sin_map,
             cos_map, kernel_mask, border_mask, interpret=False):
    map_spec = pl.BlockSpec((1, BH, W), lambda b, h: (b, h, 0))
    return pl.pallas_call(
        _tc_body,
        grid=(B, NH),
        in_specs=[pl.BlockSpec((1, 8, BH, W), lambda b, h: (b, 0, h, 0))]
        + [map_spec] * 8,
        out_specs=[
            pl.BlockSpec((1, 1, ACC_L), lambda b, h: (b, 0, 0)),
            pl.BlockSpec((1, BH, W), lambda b, h: (b, h, 0)),
        ],
        out_shape=[
            jax.ShapeDtypeStruct((B, 1, ACC_L), jnp.float32),
            jax.ShapeDtypeStruct((B, H, W), jnp.float32),
        ],
        interpret=interpret,
    )(inputs, train_mask, tr_mask, tcl_mask, radii_map, sin_map, cos_map,
      kernel_mask, border_mask)


def _sc_hist_body(ce_hbm, cnt_hbm, sum_hbm, data_v, hc_v, hs_v, oc_v, os_v):
    c = lax.axis_index("c")
    s = lax.axis_index("s")
    wid = s * 2 + c
    base = wid * CHUNK
    pltpu.sync_copy(ce_hbm.at[pl.ds(base, CHUNK)], data_v)

    zero16 = jnp.zeros((LANES,), jnp.float32)

    def zbody(i, carry):
        hc_v[pl.ds(i * LANES, LANES)] = zero16
        hs_v[pl.ds(i * LANES, LANES)] = zero16
        return carry

    lax.fori_loop(0, (LANES * NBINS) // LANES, zbody, 0)

    lane_off = lax.iota(jnp.int32, LANES) * NBINS
    ones = jnp.ones((LANES,), jnp.float32)

    def body(i, carry):
        v = data_v[pl.ds(i * LANES, LANES)]
        msk = v >= 0.0
        bi = jnp.minimum((jnp.maximum(v, 0.0) * SCALE).astype(jnp.int32),
                         NBINS - 1)
        idx = lane_off + bi
        plsc.addupdate_scatter(hc_v, [idx], ones, mask=msk)
        plsc.addupdate_scatter(hs_v, [idx], v, mask=msk)
        return carry

    lax.fori_loop(0, CHUNK // LANES, body, 0)

    def rbody(j, carry):
        acc_c = zero16
        acc_s = zero16
        for l in range(LANES):
            acc_c = acc_c + hc_v[pl.ds(l * NBINS + j * LANES, LANES)]
            acc_s = acc_s + hs_v[pl.ds(l * NBINS + j * LANES, LANES)]
        oc_v[pl.ds(j * LANES, LANES)] = acc_c
        os_v[pl.ds(j * LANES, LANES)] = acc_s
        return carry

    lax.fori_loop(0, NBINS // LANES, rbody, 0)

    pltpu.sync_copy(oc_v, cnt_hbm.at[wid])
    pltpu.sync_copy(os_v, sum_hbm.at[wid])


@functools.cache
def _sc_hist():
    return pl.kernel(
        _sc_hist_body,
        out_type=[
            jax.ShapeDtypeStruct((NW, NBINS), jnp.float32),
            jax.ShapeDtypeStruct((NW, NBINS), jnp.float32),
        ],
        mesh=plsc.VectorSubcoreMesh(core_axis_name="c",
                                    subcore_axis_name="s"),
        compiler_params=pltpu.CompilerParams(needs_layout_passes=False),
        scratch_types=[
            pltpu.VMEM((CHUNK,), jnp.float32),
            pltpu.VMEM((LANES * NBINS,), jnp.float32),
            pltpu.VMEM((LANES * NBINS,), jnp.float32),
            pltpu.VMEM((NBINS,), jnp.float32),
            pltpu.VMEM((NBINS,), jnp.float32),
        ],
    )


def _assemble(acc, cnt_rows, sum_rows):
    acc = acc.reshape(B, ACC_L)
    tot = jnp.sum(acc, axis=0)
    n_pos = tot[0]
    loss_pos = tot[1]
    n_neg_tot = tot[2]
    sum_tcl = tot[3]
    n_tclpos = tot[4]
    l1r, l1s, l1c = tot[5], tot[6], tot[7]
    da = acc[:, 8]
    db = acc[:, 9] + 0.001
    dc = acc[:, 10] + 0.001

    cnt = jnp.sum(cnt_rows, axis=0)
    sm = jnp.sum(sum_rows, axis=0)

    k = jnp.where(n_pos > 0, jnp.minimum(n_neg_tot, 3.0 * n_pos), 100.0)
    cum_incl = jnp.cumsum(cnt)
    total_cnt = cum_incl[-1]
    cum_above = total_cnt - cum_incl
    r = jnp.clip(k - cum_above, 0.0, cnt)
    contrib = jnp.where(r >= cnt, sm,
                        r * (sm / jnp.maximum(cnt, 1.0)))
    top_sum = jnp.sum(contrib) + jnp.maximum(k - total_cnt, 0.0) * (-1e30)

    loss_tr = (loss_pos + top_sum) / (n_pos + k)
    loss_tcl = sum_tcl / (n_pos + 1e-6)
    loss_radii = l1r / (n_tclpos + 1e-6)
    loss_sin = l1s / (n_tclpos + 1e-6)
    loss_cos = l1c / (n_tclpos + 1e-6)
    loss_kernel = jnp.mean(1.0 - 2.0 * da / (db + dc))
    return loss_tr + loss_tcl + loss_radii + loss_sin + loss_cos + loss_kernel


def kernel(inputs, train_mask, tr_mask, tcl_mask, radii_map, sin_map,
           cos_map, kernel_mask, border_mask):
    acc, ce_neg = _tc_call(inputs, train_mask, tr_mask, tcl_mask, radii_map,
                           sin_map, cos_map, kernel_mask, border_mask)
    cnt_rows, sum_rows = _sc_hist()(ce_neg.reshape(-1))
    return _assemble(acc, cnt_rows, sum_rows)
